# Initial kernel scaffold; baseline (speedup 1.0000x reference)
import jax
import jax.numpy as jnp
from jax.experimental import pallas as pl


def kernel(key_cache, block_tables, context_lens, positions):
    raise NotImplementedError("write your pallas kernel here")


# speedup vs baseline: 3.7512x; 3.7512x over previous
import functools
import math

import jax
import jax.numpy as jnp
from jax import lax
from jax.experimental import pallas as pl
from jax.experimental.pallas import tpu as pltpu
from jax.experimental.pallas import tpu_sc as plsc

_SINK_SIZE = 16
_SLIDING_WINDOW = 4080
_NUM_KV_HEADS = 8
_HEAD_SIZE = 128
_BLOCK_SIZE = 16
_X = 8
_NUM_BLOCKS = 2048
_BATCH = 64
_ROPE_BASE = 10000.0

_CACHE_SIZE = float(_SLIDING_WINDOW + _SINK_SIZE)
_ROW = _NUM_KV_HEADS * (_HEAD_SIZE // _X) * _BLOCK_SIZE * _X
_HALF = _HEAD_SIZE // 2
_NC = 2
_NS = 16
_NW = _NC * _NS
_SLOTS_PER_W = _BATCH // _NW


def _tables_body(btc_ref, btr_ref, posr_ref, cos_ref, sin_ref, enc_ref):
    btc = btc_ref[...]
    btr = btr_ref[...]
    posr = posr_ref[...]

    eq = btc == btr
    jidx = lax.broadcasted_iota(jnp.int32, (_BATCH, _BATCH), 1)
    firstj = jnp.min(jnp.where(eq, jidx, _BATCH), axis=1, keepdims=True)
    iidx = lax.broadcasted_iota(jnp.int32, (_BATCH, 1), 0)
    is_first = firstj == iidx

    theta = jnp.maximum(posr.astype(jnp.float32) - _CACHE_SIZE, 0.0)
    angle = jnp.sum(
        jnp.where(eq, jnp.broadcast_to(theta, (_BATCH, _BATCH)), 0.0),
        axis=1, keepdims=True)

    lane = lax.broadcasted_iota(jnp.int32, (_BATCH, _HEAD_SIZE), 1)
    f = (lane // 16) * 8 + (lane % 16) % 8
    inv_freq = jnp.exp(
        f.astype(jnp.float32) * (-2.0 * math.log(_ROPE_BASE) / _HEAD_SIZE))
    ang = angle * inv_freq
    cos_ref[...] = jnp.cos(ang)
    sin_ref[...] = jnp.sin(ang)

    proc = jnp.logical_and(is_first, angle > 0.0)
    enc = jnp.where(proc, btc, -1)
    enc_ref[...] = jnp.broadcast_to(enc, (_BATCH, 16))


def _make_tables(interpret=False):
    return pl.pallas_call(
        _tables_body,
        out_shape=(
            jax.ShapeDtypeStruct((_BATCH, _HEAD_SIZE), jnp.float32),
            jax.ShapeDtypeStruct((_BATCH, _HEAD_SIZE), jnp.float32),
            jax.ShapeDtypeStruct((_BATCH, 16), jnp.int32),
        ),
        interpret=interpret,
    )


def _sc_body(cache_ref, cos_hbm, sin_hbm, enc_hbm,
             row_v, cos_v, sin_v, enc_v, sem):
    cid = lax.axis_index("c")
    sid = lax.axis_index("s")
    wid = sid * _NC + cid
    for k in range(_SLOTS_PER_W):
        slot = wid * _SLOTS_PER_W + k
        pltpu.sync_copy(enc_hbm.at[pl.ds(slot, 1)], enc_v)
        blk = jnp.max(enc_v[0, :])

        @pl.when(blk >= 0)
        def _():
            pltpu.async_copy(cache_ref.at[pl.ds(blk, 1)], row_v, sem).wait()
            pltpu.sync_copy(cos_hbm.at[pl.ds(slot, 1)], cos_v)
            pltpu.sync_copy(sin_hbm.at[pl.ds(slot, 1)], sin_v)

            def body(hd, carry):
                h = hd // _X
                dx = hd - h * _X
                cbase = dx * 16
                off1 = (h * 16 + dx) * 128
                off2 = off1 + 1024
                c = cos_v[0, pl.ds(cbase, 16)]
                s = sin_v[0, pl.ds(cbase, 16)]
                for v in range(8):
                    o1 = off1 + v * 16
                    o2 = off2 + v * 16
                    k1 = row_v[0, pl.ds(o1, 16)]
                    k2 = row_v[0, pl.ds(o2, 16)]
                    row_v[0, pl.ds(o1, 16)] = k1 * c - k2 * s
                    row_v[0, pl.ds(o2, 16)] = k2 * c + k1 * s
                return carry

            lax.fori_loop(0, _NUM_KV_HEADS * _X, body, 0)
            pltpu.async_copy(row_v, cache_ref.at[pl.ds(blk, 1)], sem).wait()


def _make_sc_apply(interpret=False):
    mesh = plsc.VectorSubcoreMesh(
        core_axis_name="c", subcore_axis_name="s",
        num_cores=_NC, num_subcores=_NS)
    return pl.kernel(
        _sc_body,
        out_type=(),
        mesh=mesh,
        compiler_params=pltpu.CompilerParams(needs_layout_passes=False),
        scratch_types=[
            pltpu.VMEM((1, _ROW), jnp.float32),
            pltpu.VMEM((1, _HEAD_SIZE), jnp.float32),
            pltpu.VMEM((1, _HEAD_SIZE), jnp.float32),
            pltpu.VMEM((1, 16), jnp.int32),
            pltpu.SemaphoreType.DMA,
        ],
        interpret=interpret,
    )


def _kernel_impl(key_cache, block_tables, context_lens, positions,
                 interpret=False):
    del context_lens
    shape = key_cache.shape
    cache2 = key_cache.reshape(_NUM_BLOCKS, _ROW)
    btc = block_tables[:, :1]
    btr = btc.reshape(1, _BATCH)
    posr = positions.reshape(1, _BATCH)
    cos_t, sin_t, enc = _make_tables(interpret)(btc, btr, posr)
    cache_ref = jax.new_ref(cache2)
    _make_sc_apply(interpret)(cache_ref, cos_t, sin_t, enc)
    return cache_ref[...].reshape(shape)


def kernel(key_cache, block_tables, context_lens, positions):
    return _kernel_impl(key_cache, block_tables, context_lens, positions)
